# baseline (device time: 206064 ns/iter reference)
import jax
import jax.numpy as jnp
from jax import lax
from jax.experimental import pallas as pl
from jax.experimental.pallas import tpu as pltpu

N_DEV = 16
B = 2
SQ = 256
SKV = 256
HQ = 4
DH = 64
DMODEL = 512
BLK = 64
BH = B * HQ


def kernel(x, Wq, K_ext, V_ext, Wo):
    def body(x_ref, wq_ref, k_ref, v_ref, wo_ref, out_ref,
             k_all, v_all, k_ssems, k_rsems, v_ssems, v_rsems):
        my_pos = lax.axis_index("i")
        left = lax.rem(my_pos - 1 + N_DEV, N_DEV)
        right = lax.rem(my_pos + 1, N_DEV)

        barrier_sem = pltpu.get_barrier_semaphore()
        for nbr in (left, right):
            pl.semaphore_signal(
                barrier_sem, inc=1,
                device_id=(nbr,), device_id_type=pl.DeviceIdType.MESH,
            )
        pl.semaphore_wait(barrier_sem, 2)

        k_all[0] = jnp.transpose(
            k_ref[...].astype(jnp.bfloat16), (0, 2, 1, 3)).reshape(
                BH, SKV, DH)
        v_all[0] = jnp.transpose(
            v_ref[...].astype(jnp.bfloat16), (0, 2, 1, 3)).reshape(
                BH, SKV, DH)

        def hop(h):
            ops = []
            for buf, ssems, rsems in ((k_all, k_ssems, k_rsems),
                                      (v_all, v_ssems, v_rsems)):
                r = pltpu.make_async_remote_copy(
                    src_ref=buf.at[h],
                    dst_ref=buf.at[h + 1],
                    send_sem=ssems.at[h],
                    recv_sem=rsems.at[h],
                    device_id=(right,),
                    device_id_type=pl.DeviceIdType.MESH,
                )
                ops.append(r)
            return ops

        wq = wq_ref[...].astype(jnp.bfloat16)
        q_all = jnp.stack([
            jnp.transpose(
                (jnp.dot(x_ref[b].astype(jnp.bfloat16), wq,
                         preferred_element_type=jnp.float32) * 0.125)
                .astype(jnp.bfloat16).reshape(SQ, HQ, DH),
                (1, 0, 2))
            for b in range(B)
        ]).reshape(BH, SQ, DH)

        li = lax.broadcasted_iota(jnp.int32, (SQ, SKV), 0)
        lj = lax.broadcasted_iota(jnp.int32, (SQ, SKV), 1)
        diag_bias = jnp.where((lj // BLK) <= (li // BLK),
                              0.0, -1e9).astype(jnp.float32)

        def fold(s, num, den, bias):
            k_c = k_all[s]
            v_c = v_all[s]
            scores = lax.dot_general(
                q_all, k_c,
                dimension_numbers=(((2,), (2,)), ((0,), (0,))),
                preferred_element_type=jnp.float32,
            )
            w = jnp.exp(scores + bias)
            num = num + lax.dot_general(
                w.astype(jnp.bfloat16), v_c,
                dimension_numbers=(((2,), (1,)), ((0,), (0,))),
                preferred_element_type=jnp.float32,
            )
            den = den + jnp.sum(w, axis=-1, keepdims=True)
            return num, den

        ops = hop(0)
        for r in ops:
            r.start()
        num = jnp.zeros((BH, SQ, DH), jnp.float32)
        den = jnp.zeros((BH, SQ, 1), jnp.float32)
        num, den = fold(0, num, den, diag_bias[None])

        prev_ops = ops
        for s in range(1, N_DEV):
            for r in prev_ops:
                r.wait_recv()
            if s < N_DEV - 1:
                ops = hop(s)
                for r in ops:
                    r.start()
            bias = jnp.where(s <= my_pos, 0.0, -1e9).astype(jnp.float32)
            num, den = fold(s, num, den, bias)
            for r in prev_ops:
                r.wait_send()
            prev_ops = ops

        ctx = (num / den).reshape(B, HQ, SQ, DH)
        wo = wo_ref[...].astype(jnp.bfloat16)
        for b in range(B):
            ctx_b = jnp.transpose(ctx[b], (1, 0, 2)).reshape(SQ, HQ * DH)
            out_ref[b] = jnp.dot(ctx_b.astype(jnp.bfloat16), wo,
                                 preferred_element_type=jnp.float32)

    return pl.pallas_call(
        body,
        out_shape=jax.ShapeDtypeStruct((B, SQ, DMODEL), jnp.float32),
        in_specs=[pl.BlockSpec(memory_space=pltpu.VMEM)] * 5,
        out_specs=pl.BlockSpec(memory_space=pltpu.VMEM),
        scratch_shapes=[
            pltpu.VMEM((N_DEV, BH, SKV, DH), jnp.bfloat16),
            pltpu.VMEM((N_DEV, BH, SKV, DH), jnp.bfloat16),
            pltpu.SemaphoreType.DMA((N_DEV - 1,)),
            pltpu.SemaphoreType.DMA((N_DEV - 1,)),
            pltpu.SemaphoreType.DMA((N_DEV - 1,)),
            pltpu.SemaphoreType.DMA((N_DEV - 1,)),
        ],
        compiler_params=pltpu.CompilerParams(collective_id=0),
    )(x, Wq, K_ext, V_ext, Wo)


# device time: 126803 ns/iter; 1.6251x vs baseline; 1.6251x over previous
import jax
import jax.numpy as jnp
from jax import lax
from jax.experimental import pallas as pl
from jax.experimental.pallas import tpu as pltpu

N_DEV = 16
B = 2
SQ = 256
SKV = 256
HQ = 4
DH = 64
DMODEL = 512
BLK = 64
BH = B * HQ
R_HOPS = 8
L_HOPS = 7


def kernel(x, Wq, K_ext, V_ext, Wo):
    def body(x_ref, wq_ref, k_ref, v_ref, wo_ref, out_ref,
             rbuf, lbuf, r_ssems, r_rsems, l_ssems, l_rsems):
        my_pos = lax.axis_index("i")
        left = lax.rem(my_pos - 1 + N_DEV, N_DEV)
        right = lax.rem(my_pos + 1, N_DEV)

        barrier_sem = pltpu.get_barrier_semaphore()
        for nbr in (left, right):
            pl.semaphore_signal(
                barrier_sem, inc=1,
                device_id=(nbr,), device_id_type=pl.DeviceIdType.MESH,
            )
        pl.semaphore_wait(barrier_sem, 2)

        kt = jnp.transpose(
            k_ref[...].astype(jnp.bfloat16), (0, 2, 1, 3)).reshape(
                BH, SKV, DH)
        vt = jnp.transpose(
            v_ref[...].astype(jnp.bfloat16), (0, 2, 1, 3)).reshape(
                BH, SKV, DH)
        rbuf[0, 0] = kt
        rbuf[0, 1] = vt
        lbuf[0, 0] = kt
        lbuf[0, 1] = vt

        def hop(buf, ssems, rsems, h, dst):
            return pltpu.make_async_remote_copy(
                src_ref=buf.at[h],
                dst_ref=buf.at[h + 1],
                send_sem=ssems.at[h],
                recv_sem=rsems.at[h],
                device_id=(dst,),
                device_id_type=pl.DeviceIdType.MESH,
            )

        wq = wq_ref[...].astype(jnp.bfloat16)
        q_all = jnp.stack([
            jnp.transpose(
                (jnp.dot(x_ref[b].astype(jnp.bfloat16), wq,
                         preferred_element_type=jnp.float32) * 0.125)
                .astype(jnp.bfloat16).reshape(SQ, HQ, DH),
                (1, 0, 2))
            for b in range(B)
        ]).reshape(BH, SQ, DH)

        li = lax.broadcasted_iota(jnp.int32, (SQ, SKV), 0)
        lj = lax.broadcasted_iota(jnp.int32, (SQ, SKV), 1)
        diag_bias = jnp.where((lj // BLK) <= (li // BLK),
                              0.0, -1e9).astype(jnp.float32)

        def fold(buf, s, num, den, bias):
            k_c = buf[s, 0]
            v_c = buf[s, 1]
            scores = lax.dot_general(
                q_all, k_c,
                dimension_numbers=(((2,), (2,)), ((0,), (0,))),
                preferred_element_type=jnp.float32,
            )
            w = jnp.exp(scores + bias)
            num = num + lax.dot_general(
                w.astype(jnp.bfloat16), v_c,
                dimension_numbers=(((2,), (1,)), ((0,), (0,))),
                preferred_element_type=jnp.float32,
            )
            den = den + jnp.sum(w, axis=-1, keepdims=True)
            return num, den

        r_op = hop(rbuf, r_ssems, r_rsems, 0, right)
        l_op = hop(lbuf, l_ssems, l_rsems, 0, left)
        r_op.start()
        l_op.start()
        num = jnp.zeros((BH, SQ, DH), jnp.float32)
        den = jnp.zeros((BH, SQ, 1), jnp.float32)
        num, den = fold(rbuf, 0, num, den, diag_bias[None])

        prev_r, prev_l = r_op, l_op
        for s in range(1, R_HOPS + 1):
            prev_r.wait_recv()
            if s <= L_HOPS:
                prev_l.wait_recv()
            if s < R_HOPS:
                r_op = hop(rbuf, r_ssems, r_rsems, s, right)
                r_op.start()
            if s < L_HOPS:
                l_op = hop(lbuf, l_ssems, l_rsems, s, left)
                l_op.start()
            r_bias = jnp.where(s <= my_pos, 0.0, -1e9).astype(jnp.float32)
            num, den = fold(rbuf, s, num, den, r_bias)
            if s <= L_HOPS:
                l_bias = jnp.where(my_pos + s >= N_DEV, 0.0,
                                   -1e9).astype(jnp.float32)
                num, den = fold(lbuf, s, num, den, l_bias)
            prev_r.wait_send()
            if s <= L_HOPS:
                prev_l.wait_send()
            prev_r, prev_l = r_op, l_op

        ctx = (num / den).reshape(B, HQ, SQ, DH)
        wo = wo_ref[...].astype(jnp.bfloat16)
        for b in range(B):
            ctx_b = jnp.transpose(ctx[b], (1, 0, 2)).reshape(SQ, HQ * DH)
            out_ref[b] = jnp.dot(ctx_b.astype(jnp.bfloat16), wo,
                                 preferred_element_type=jnp.float32)

    return pl.pallas_call(
        body,
        out_shape=jax.ShapeDtypeStruct((B, SQ, DMODEL), jnp.float32),
        in_specs=[pl.BlockSpec(memory_space=pltpu.VMEM)] * 5,
        out_specs=pl.BlockSpec(memory_space=pltpu.VMEM),
        scratch_shapes=[
            pltpu.VMEM((R_HOPS + 1, 2, BH, SKV, DH), jnp.bfloat16),
            pltpu.VMEM((L_HOPS + 1, 2, BH, SKV, DH), jnp.bfloat16),
            pltpu.SemaphoreType.DMA((R_HOPS,)),
            pltpu.SemaphoreType.DMA((R_HOPS,)),
            pltpu.SemaphoreType.DMA((L_HOPS,)),
            pltpu.SemaphoreType.DMA((L_HOPS,)),
        ],
        compiler_params=pltpu.CompilerParams(collective_id=0),
    )(x, Wq, K_ext, V_ext, Wo)


# device time: 104560 ns/iter; 1.9708x vs baseline; 1.2127x over previous
import jax
import jax.numpy as jnp
from jax import lax
from jax.experimental import pallas as pl
from jax.experimental.pallas import tpu as pltpu

N_DEV = 16
B = 2
SQ = 256
SKV = 256
HQ = 4
DH = 64
DMODEL = 512
BLK = 64
BH = B * HQ
R_HOPS = 8
L_HOPS = 7


def kernel(x, Wq, K_ext, V_ext, Wo):
    def body(x_ref, wq_ref, k_ref, v_ref, wo_ref, out_ref,
             rbuf, lbuf,
             rk_ss, rk_rs, rv_ss, rv_rs,
             lk_ss, lk_rs, lv_ss, lv_rs):
        my_pos = lax.axis_index("i")
        left = lax.rem(my_pos - 1 + N_DEV, N_DEV)
        right = lax.rem(my_pos + 1, N_DEV)

        barrier_sem = pltpu.get_barrier_semaphore()
        for nbr in (left, right):
            pl.semaphore_signal(
                barrier_sem, inc=1,
                device_id=(nbr,), device_id_type=pl.DeviceIdType.MESH,
            )
        pl.semaphore_wait(barrier_sem, 2)

        kt = jnp.transpose(
            k_ref[...].astype(jnp.bfloat16), (0, 2, 1, 3)).reshape(
                BH, SKV, DH)
        vt = jnp.transpose(
            v_ref[...].astype(jnp.bfloat16), (0, 2, 1, 3)).reshape(
                BH, SKV, DH)
        rbuf[0, 0] = kt
        rbuf[0, 1] = vt
        lbuf[0, 0] = kt
        lbuf[0, 1] = vt

        def piece(buf, ssems, rsems, h, part, dst):
            return pltpu.make_async_remote_copy(
                src_ref=buf.at[h, part],
                dst_ref=buf.at[h + 1, part],
                send_sem=ssems.at[h],
                recv_sem=rsems.at[h],
                device_id=(dst,),
                device_id_type=pl.DeviceIdType.MESH,
            )

        def r_pieces(h):
            return (piece(rbuf, rk_ss, rk_rs, h, 0, right),
                    piece(rbuf, rv_ss, rv_rs, h, 1, right))

        def l_pieces(h):
            return (piece(lbuf, lk_ss, lk_rs, h, 0, left),
                    piece(lbuf, lv_ss, lv_rs, h, 1, left))

        wq = wq_ref[...].astype(jnp.bfloat16)
        q_all = jnp.stack([
            jnp.transpose(
                (jnp.dot(x_ref[b].astype(jnp.bfloat16), wq,
                         preferred_element_type=jnp.float32) * 0.125)
                .astype(jnp.bfloat16).reshape(SQ, HQ, DH),
                (1, 0, 2))
            for b in range(B)
        ]).reshape(BH, SQ, DH)

        li = lax.broadcasted_iota(jnp.int32, (SQ, SKV), 0)
        lj = lax.broadcasted_iota(jnp.int32, (SQ, SKV), 1)
        diag_bias = jnp.where((lj // BLK) <= (li // BLK),
                              0.0, -1e9).astype(jnp.float32)

        def fold(buf, s, num, den, bias):
            k_c = buf[s, 0]
            v_c = buf[s, 1]
            scores = lax.dot_general(
                q_all, k_c,
                dimension_numbers=(((2,), (2,)), ((0,), (0,))),
                preferred_element_type=jnp.float32,
            )
            w = jnp.exp(scores + bias)
            num = num + lax.dot_general(
                w.astype(jnp.bfloat16), v_c,
                dimension_numbers=(((2,), (1,)), ((0,), (0,))),
                preferred_element_type=jnp.float32,
            )
            den = den + jnp.sum(w, axis=-1, keepdims=True)
            return num, den

        r_ops = r_pieces(0)
        l_ops = l_pieces(0)
        for r in r_ops + l_ops:
            r.start()
        num = jnp.zeros((BH, SQ, DH), jnp.float32)
        den = jnp.zeros((BH, SQ, 1), jnp.float32)
        num, den = fold(rbuf, 0, num, den, diag_bias[None])

        prev_r, prev_l = r_ops, l_ops
        for s in range(1, R_HOPS + 1):
            nr = r_pieces(s) if s < R_HOPS else None
            nl = l_pieces(s) if s < L_HOPS else None
            prev_r[0].wait_recv()
            if nr:
                nr[0].start()
            if s <= L_HOPS:
                prev_l[0].wait_recv()
                if nl:
                    nl[0].start()
            prev_r[1].wait_recv()
            if nr:
                nr[1].start()
            if s <= L_HOPS:
                prev_l[1].wait_recv()
                if nl:
                    nl[1].start()
            r_bias = jnp.where(s <= my_pos, 0.0, -1e9).astype(jnp.float32)
            num, den = fold(rbuf, s, num, den, r_bias)
            if s <= L_HOPS:
                l_bias = jnp.where(my_pos + s >= N_DEV, 0.0,
                                   -1e9).astype(jnp.float32)
                num, den = fold(lbuf, s, num, den, l_bias)
            for r in prev_r:
                r.wait_send()
            if s <= L_HOPS:
                for r in prev_l:
                    r.wait_send()
            prev_r, prev_l = nr, nl

        ctx = (num / den).reshape(B, HQ, SQ, DH)
        wo = wo_ref[...].astype(jnp.bfloat16)
        for b in range(B):
            ctx_b = jnp.transpose(ctx[b], (1, 0, 2)).reshape(SQ, HQ * DH)
            out_ref[b] = jnp.dot(ctx_b.astype(jnp.bfloat16), wo,
                                 preferred_element_type=jnp.float32)

    return pl.pallas_call(
        body,
        out_shape=jax.ShapeDtypeStruct((B, SQ, DMODEL), jnp.float32),
        in_specs=[pl.BlockSpec(memory_space=pltpu.VMEM)] * 5,
        out_specs=pl.BlockSpec(memory_space=pltpu.VMEM),
        scratch_shapes=[
            pltpu.VMEM((R_HOPS + 1, 2, BH, SKV, DH), jnp.bfloat16),
            pltpu.VMEM((L_HOPS + 1, 2, BH, SKV, DH), jnp.bfloat16),
            pltpu.SemaphoreType.DMA((R_HOPS,)),
            pltpu.SemaphoreType.DMA((R_HOPS,)),
            pltpu.SemaphoreType.DMA((R_HOPS,)),
            pltpu.SemaphoreType.DMA((R_HOPS,)),
            pltpu.SemaphoreType.DMA((L_HOPS,)),
            pltpu.SemaphoreType.DMA((L_HOPS,)),
            pltpu.SemaphoreType.DMA((L_HOPS,)),
            pltpu.SemaphoreType.DMA((L_HOPS,)),
        ],
        compiler_params=pltpu.CompilerParams(collective_id=0),
    )(x, Wq, K_ext, V_ext, Wo)
